# baseline (speedup 1.0000x reference)
import functools

import jax
import jax.numpy as jnp
from jax import lax
from jax.experimental import pallas as pl
from jax.experimental.pallas import tpu as pltpu
from jax.experimental.pallas import tpu_sc as plsc

_F = 26
_V = 100000
_VP = 100096
_E = 16
_B = 16384
_C = 13

_NC, _NS, _L = 2, 16, 16
_NW = _NC * _NS
_BW = _B // _NW

_VT = 781
_NU = _F * _VT
_FSTRIDE = _VP * _E


@functools.partial(
    pl.kernel,
    out_type=jax.ShapeDtypeStruct((_F * _FSTRIDE,), jnp.float32),
    mesh=plsc.VectorSubcoreMesh(core_axis_name="c", subcore_axis_name="s"),
    scratch_types=[
        pltpu.VMEM((4, 16, 512), jnp.float32),
        pltpu.VMEM((4 * 8192,), jnp.float32),
        pltpu.SemaphoreType.DMA,
        pltpu.SemaphoreType.DMA,
    ],
    compiler_params=pltpu.CompilerParams(
        use_tc_tiling_on_sc=True, needs_layout_passes=False
    ),
)
def _relayout_kernel(tt_hbm, l32_hbm, pk_hbm, tile_v, rows_v, rsem, wsem):
    wid = lax.axis_index("s") * _NC + lax.axis_index("c")
    nq = _F * 195
    q_lo = wid * nq // _NW
    q_hi = (wid + 1) * nq // _NW

    tgts = [lax.iota(jnp.int32, _L) * _E + e for e in range(_E)]
    nbuf = 4

    def issue_read(q, buf):
        f = q // 195
        vq = q % 195
        pltpu.async_copy(
            tt_hbm.at[pl.ds(f, 1), :, pl.ds(vq * 512, 512)],
            tile_v.at[pl.ds(buf, 1)],
            rsem,
        )

    def drain_read():
        pltpu.make_async_copy(
            tt_hbm.at[pl.ds(0, 1), :, pl.ds(0, 512)],
            tile_v.at[pl.ds(0, 1)],
            rsem,
        ).wait()

    def drain_write():
        pltpu.make_async_copy(
            rows_v.at[pl.ds(0, 8192)], pk_hbm.at[pl.ds(0, 8192)], wsem
        ).wait()

    for k in range(nbuf):
        issue_read(q_lo + k, k)

    def round_(r, carry):
        q0 = q_lo + r * nbuf
        for buf in range(nbuf):
            q = q0 + buf

            @pl.when(q < q_hi)
            def _(q=q, buf=buf):
                drain_read()

                @pl.when(q - q_lo >= nbuf)
                def _():
                    drain_write()

                rbase = buf * 8192
                for j in range(512 // _L):
                    win = rows_v.at[pl.ds(rbase + j * _L * _E, 256)]
                    for e in range(_E):
                        vals = tile_v[buf, e, pl.ds(j * _L, _L)]
                        plsc.store_scatter(win, [tgts[e]], vals)

                @pl.when(q + nbuf < q_hi)
                def _():
                    issue_read(q + nbuf, buf)

                f = q // 195
                vq = q % 195
                pltpu.async_copy(
                    rows_v.at[pl.ds(rbase, 8192)],
                    pk_hbm.at[pl.ds(f * _FSTRIDE + vq * 8192, 8192)],
                    wsem,
                )
        return carry

    lax.fori_loop(0, (q_hi - q_lo + nbuf - 1) // nbuf, round_, 0)
    for _k in range(nbuf):
        drain_write()

    @pl.when(wid < _F)
    def _():
        pltpu.sync_copy(
            tt_hbm.at[pl.ds(wid, 1), :, pl.ds(780 * 128, 128)],
            tile_v.at[pl.ds(0, 1), :, pl.ds(0, 128)],
        )
        for j in range(128 // _L):
            win = rows_v.at[pl.ds(j * _L * _E, 256)]
            for e in range(_E):
                vals = tile_v[0, e, pl.ds(j * _L, _L)]
                plsc.store_scatter(win, [tgts[e]], vals)
        pltpu.sync_copy(
            rows_v.at[pl.ds(0, 2048)],
            pk_hbm.at[pl.ds(wid * _FSTRIDE + 780 * 2048, 2048)],
        )
        pltpu.sync_copy(
            l32_hbm.at[pl.ds(wid * 512, 512)],
            pk_hbm.at[pl.ds(wid * _FSTRIDE + (_V - 32) * _E, 512)],
        )


@functools.partial(
    pl.kernel,
    out_type=jax.ShapeDtypeStruct((_F * _B, _E), jnp.float32),
    mesh=plsc.VectorSubcoreMesh(core_axis_name="c", subcore_axis_name="s"),
    scratch_types=[
        pltpu.VMEM((_BW,), jnp.int32),
        pltpu.VMEM((_BW,), jnp.int32),
        pltpu.VMEM((_BW,), jnp.int32),
        pltpu.VMEM((_BW,), jnp.int32),
        pltpu.VMEM((_BW, _E), jnp.float32),
        pltpu.VMEM((_BW, _E), jnp.float32),
        pltpu.SemaphoreType.DMA,
        pltpu.SemaphoreType.DMA,
    ],
    compiler_params=pltpu.CompilerParams(use_tc_tiling_on_sc=False),
)
def _gather_kernel(
    xcft_hbm, pk_hbm, out_hbm, idx0, idx1, oidx0, oidx1, rows0, rows1, gsem, wsem
):
    wid = lax.axis_index("s") * _NC + lax.axis_index("c")
    base = wid * _BW
    idxs = (idx0, idx1)
    oidxs = (oidx0, oidx1)
    rows = (rows0, rows1)
    row0 = (base + lax.iota(jnp.int32, _L)) * _F

    def load_and_fire(f):
        iv = idxs[f % 2]
        ov = oidxs[f % 2]
        pltpu.sync_copy(xcft_hbm.at[pl.ds(f * _B + base, _BW)], iv)

        def addoff(i, carry2):
            sl = pl.ds(i * _L, _L)
            iv[sl] = iv[sl] + f * _VP
            ov[sl] = row0 + (i * _L * _F + f)
            return carry2

        lax.fori_loop(0, _BW // _L, addoff, 0, unroll=4)
        pltpu.async_copy(pk_hbm.at[iv], rows[f % 2], gsem)

    def drain_write():
        pltpu.make_async_copy(rows0, out_hbm.at[oidx0], wsem).wait()

    load_and_fire(0)
    for f in range(_F):
        if f + 1 < _F:
            if f >= 1:
                drain_write()
            load_and_fire(f + 1)
        pltpu.make_async_copy(pk_hbm.at[idxs[f % 2]], rows[f % 2], gsem).wait()
        pltpu.async_copy(rows[f % 2], out_hbm.at[oidxs[f % 2]], wsem)
    drain_write()
    drain_write()


def kernel(x_cont, x_cat, tables):
    tt = jnp.transpose(tables, (0, 2, 1))
    l32 = tables[:, _V - 32:, :].reshape(_F * 32 * _E)
    pk = _relayout_kernel(tt, l32)
    xcft = jnp.transpose(x_cat, (1, 0)).reshape(_F * _B)
    cat_bm = _gather_kernel(xcft, pk.reshape(_F * _VP, _E))
    return jnp.concatenate([x_cont, cat_bm.reshape(_B, _F * _E)], axis=1)


# speedup vs baseline: 1.1965x; 1.1965x over previous
import functools

import jax
import jax.numpy as jnp
from jax import lax
from jax.experimental import pallas as pl
from jax.experimental.pallas import tpu as pltpu
from jax.experimental.pallas import tpu_sc as plsc

_F = 26
_V = 100000
_VP = 100096
_E = 16
_B = 16384
_C = 13

_NC, _NS, _L = 2, 16, 16
_NW = _NC * _NS
_BW = _B // _NW

_VT = 781
_NU = _F * _VT
_FSTRIDE = _VP * _E


@functools.partial(
    pl.kernel,
    out_type=jax.ShapeDtypeStruct((_F * _FSTRIDE,), jnp.float32),
    mesh=plsc.VectorSubcoreMesh(core_axis_name="c", subcore_axis_name="s"),
    scratch_types=[
        pltpu.VMEM((4, 16, 512), jnp.float32),
        pltpu.VMEM((4 * 8192,), jnp.float32),
        pltpu.SemaphoreType.DMA,
        pltpu.SemaphoreType.DMA,
    ],
    compiler_params=pltpu.CompilerParams(
        use_tc_tiling_on_sc=True, needs_layout_passes=False
    ),
)
def _relayout_kernel(tt_hbm, l32_hbm, pk_hbm, tile_v, rows_v, rsem, wsem):
    wid = lax.axis_index("s") * _NC + lax.axis_index("c")
    nq = _F * 195
    q_lo = wid * nq // _NW
    q_hi = (wid + 1) * nq // _NW

    tgts = [lax.iota(jnp.int32, _L) * _E + e for e in range(_E)]
    nbuf = 4

    def issue_read(q, buf):
        f = q // 195
        vq = q % 195
        pltpu.async_copy(
            tt_hbm.at[pl.ds(f, 1), :, pl.ds(vq * 512, 512)],
            tile_v.at[pl.ds(buf, 1)],
            rsem,
        )

    def drain_read():
        pltpu.make_async_copy(
            tt_hbm.at[pl.ds(0, 1), :, pl.ds(0, 512)],
            tile_v.at[pl.ds(0, 1)],
            rsem,
        ).wait()

    def drain_write():
        pltpu.make_async_copy(
            rows_v.at[pl.ds(0, 8192)], pk_hbm.at[pl.ds(0, 8192)], wsem
        ).wait()

    for k in range(nbuf):
        issue_read(q_lo + k, k)

    def unit(q, carry):
        buf = (q - q_lo) % nbuf
        drain_read()

        @pl.when(q - q_lo >= nbuf)
        def _():
            drain_write()

        rbase = buf * 8192
        for j in range(512 // _L):
            win = rows_v.at[pl.ds(rbase + j * _L * _E, 256)]
            for e in range(_E):
                vals = tile_v[buf, e, pl.ds(j * _L, _L)]
                plsc.store_scatter(win, [tgts[e]], vals)

        @pl.when(q + nbuf < q_hi)
        def _():
            issue_read(q + nbuf, buf)

        f = q // 195
        vq = q % 195
        pltpu.async_copy(
            rows_v.at[pl.ds(rbase, 8192)],
            pk_hbm.at[pl.ds(f * _FSTRIDE + vq * 8192, 8192)],
            wsem,
        )
        return carry

    lax.fori_loop(q_lo, q_hi, unit, 0)
    for _k in range(nbuf):
        drain_write()

    @pl.when(wid < _F)
    def _():
        pltpu.sync_copy(
            tt_hbm.at[pl.ds(wid, 1), :, pl.ds(780 * 128, 128)],
            tile_v.at[pl.ds(0, 1), :, pl.ds(0, 128)],
        )
        for j in range(128 // _L):
            win = rows_v.at[pl.ds(j * _L * _E, 256)]
            for e in range(_E):
                vals = tile_v[0, e, pl.ds(j * _L, _L)]
                plsc.store_scatter(win, [tgts[e]], vals)
        pltpu.sync_copy(
            rows_v.at[pl.ds(0, 2048)],
            pk_hbm.at[pl.ds(wid * _FSTRIDE + 780 * 2048, 2048)],
        )
        pltpu.sync_copy(
            l32_hbm.at[pl.ds(wid * 512, 512)],
            pk_hbm.at[pl.ds(wid * _FSTRIDE + (_V - 32) * _E, 512)],
        )


@functools.partial(
    pl.kernel,
    out_type=jax.ShapeDtypeStruct((_F * _B, _E), jnp.float32),
    mesh=plsc.VectorSubcoreMesh(core_axis_name="c", subcore_axis_name="s"),
    scratch_types=[
        pltpu.VMEM((_BW,), jnp.int32),
        pltpu.VMEM((_BW,), jnp.int32),
        pltpu.VMEM((_BW,), jnp.int32),
        pltpu.VMEM((_BW,), jnp.int32),
        pltpu.VMEM((_BW, _E), jnp.float32),
        pltpu.VMEM((_BW, _E), jnp.float32),
        pltpu.SemaphoreType.DMA,
        pltpu.SemaphoreType.DMA,
    ],
    compiler_params=pltpu.CompilerParams(use_tc_tiling_on_sc=False),
)
def _gather_kernel(
    xcft_hbm, pk_hbm, out_hbm, idx0, idx1, oidx0, oidx1, rows0, rows1, gsem, wsem
):
    wid = lax.axis_index("s") * _NC + lax.axis_index("c")
    base = wid * _BW
    idxs = (idx0, idx1)
    oidxs = (oidx0, oidx1)
    rows = (rows0, rows1)
    row0 = (base + lax.iota(jnp.int32, _L)) * _F

    def load_and_fire(f):
        iv = idxs[f % 2]
        ov = oidxs[f % 2]
        pltpu.sync_copy(xcft_hbm.at[pl.ds(f * _B + base, _BW)], iv)

        def addoff(i, carry2):
            sl = pl.ds(i * _L, _L)
            iv[sl] = iv[sl] + f * _VP
            ov[sl] = row0 + (i * _L * _F + f)
            return carry2

        lax.fori_loop(0, _BW // _L, addoff, 0, unroll=4)
        pltpu.async_copy(pk_hbm.at[iv], rows[f % 2], gsem)

    def drain_write():
        pltpu.make_async_copy(rows0, out_hbm.at[oidx0], wsem).wait()

    load_and_fire(0)
    for f in range(_F):
        if f + 1 < _F:
            if f >= 1:
                drain_write()
            load_and_fire(f + 1)
        pltpu.make_async_copy(pk_hbm.at[idxs[f % 2]], rows[f % 2], gsem).wait()
        pltpu.async_copy(rows[f % 2], out_hbm.at[oidxs[f % 2]], wsem)
    drain_write()
    drain_write()


def kernel(x_cont, x_cat, tables):
    tt = jnp.transpose(tables, (0, 2, 1))
    l32 = tables[:, _V - 32:, :].reshape(_F * 32 * _E)
    pk = _relayout_kernel(tt, l32)
    xcft = jnp.transpose(x_cat, (1, 0)).reshape(_F * _B)
    cat_bm = _gather_kernel(xcft, pk.reshape(_F * _VP, _E))
    return jnp.concatenate([x_cont, cat_bm.reshape(_B, _F * _E)], axis=1)


# speedup vs baseline: 1.1984x; 1.0016x over previous
import functools

import jax
import jax.numpy as jnp
from jax import lax
from jax.experimental import pallas as pl
from jax.experimental.pallas import tpu as pltpu
from jax.experimental.pallas import tpu_sc as plsc

_F = 26
_V = 100000
_VP = 100096
_E = 16
_B = 16384
_C = 13

_NC, _NS, _L = 2, 16, 16
_NW = _NC * _NS
_BW = _B // _NW

_VT = 781
_NU = _F * _VT
_FSTRIDE = _VP * _E


@functools.partial(
    pl.kernel,
    out_type=jax.ShapeDtypeStruct((_F * _FSTRIDE,), jnp.float32),
    mesh=plsc.VectorSubcoreMesh(core_axis_name="c", subcore_axis_name="s"),
    scratch_types=[
        pltpu.VMEM((6, 16, 512), jnp.float32),
        pltpu.VMEM((6 * 8192,), jnp.float32),
        pltpu.SemaphoreType.DMA,
        pltpu.SemaphoreType.DMA,
    ],
    compiler_params=pltpu.CompilerParams(
        use_tc_tiling_on_sc=True, needs_layout_passes=False
    ),
)
def _relayout_kernel(tt_hbm, l32_hbm, pk_hbm, tile_v, rows_v, rsem, wsem):
    wid = lax.axis_index("s") * _NC + lax.axis_index("c")
    nq = _F * 195
    q_lo = wid * nq // _NW
    q_hi = (wid + 1) * nq // _NW

    tgts = [lax.iota(jnp.int32, _L) * _E + e for e in range(_E)]
    nbuf = 6

    def issue_read(q, buf):
        f = q // 195
        vq = q % 195
        pltpu.async_copy(
            tt_hbm.at[pl.ds(f, 1), :, pl.ds(vq * 512, 512)],
            tile_v.at[pl.ds(buf, 1)],
            rsem,
        )

    def drain_read():
        pltpu.make_async_copy(
            tt_hbm.at[pl.ds(0, 1), :, pl.ds(0, 512)],
            tile_v.at[pl.ds(0, 1)],
            rsem,
        ).wait()

    def drain_write():
        pltpu.make_async_copy(
            rows_v.at[pl.ds(0, 8192)], pk_hbm.at[pl.ds(0, 8192)], wsem
        ).wait()

    for k in range(nbuf):
        issue_read(q_lo + k, k)

    def unit(q, carry):
        buf = (q - q_lo) % nbuf
        drain_read()

        @pl.when(q - q_lo >= nbuf)
        def _():
            drain_write()

        rbase = buf * 8192
        for j in range(512 // _L):
            win = rows_v.at[pl.ds(rbase + j * _L * _E, 256)]
            for e in range(_E):
                vals = tile_v[buf, e, pl.ds(j * _L, _L)]
                plsc.store_scatter(win, [tgts[e]], vals)

        @pl.when(q + nbuf < q_hi)
        def _():
            issue_read(q + nbuf, buf)

        f = q // 195
        vq = q % 195
        pltpu.async_copy(
            rows_v.at[pl.ds(rbase, 8192)],
            pk_hbm.at[pl.ds(f * _FSTRIDE + vq * 8192, 8192)],
            wsem,
        )
        return carry

    lax.fori_loop(q_lo, q_hi, unit, 0)
    for _k in range(nbuf):
        drain_write()

    @pl.when(wid < _F)
    def _():
        pltpu.sync_copy(
            tt_hbm.at[pl.ds(wid, 1), :, pl.ds(780 * 128, 128)],
            tile_v.at[pl.ds(0, 1), :, pl.ds(0, 128)],
        )
        for j in range(128 // _L):
            win = rows_v.at[pl.ds(j * _L * _E, 256)]
            for e in range(_E):
                vals = tile_v[0, e, pl.ds(j * _L, _L)]
                plsc.store_scatter(win, [tgts[e]], vals)
        pltpu.sync_copy(
            rows_v.at[pl.ds(0, 2048)],
            pk_hbm.at[pl.ds(wid * _FSTRIDE + 780 * 2048, 2048)],
        )
        pltpu.sync_copy(
            l32_hbm.at[pl.ds(wid * 512, 512)],
            pk_hbm.at[pl.ds(wid * _FSTRIDE + (_V - 32) * _E, 512)],
        )


@functools.partial(
    pl.kernel,
    out_type=jax.ShapeDtypeStruct((_F * _B, _E), jnp.float32),
    mesh=plsc.VectorSubcoreMesh(core_axis_name="c", subcore_axis_name="s"),
    scratch_types=[
        pltpu.VMEM((_BW,), jnp.int32),
        pltpu.VMEM((_BW,), jnp.int32),
        pltpu.VMEM((_BW,), jnp.int32),
        pltpu.VMEM((_BW,), jnp.int32),
        pltpu.VMEM((_BW, _E), jnp.float32),
        pltpu.VMEM((_BW, _E), jnp.float32),
        pltpu.SemaphoreType.DMA,
        pltpu.SemaphoreType.DMA,
    ],
    compiler_params=pltpu.CompilerParams(use_tc_tiling_on_sc=False),
)
def _gather_kernel(
    xcft_hbm, pk_hbm, out_hbm, idx0, idx1, oidx0, oidx1, rows0, rows1, gsem, wsem
):
    wid = lax.axis_index("s") * _NC + lax.axis_index("c")
    base = wid * _BW
    idxs = (idx0, idx1)
    oidxs = (oidx0, oidx1)
    rows = (rows0, rows1)
    row0 = (base + lax.iota(jnp.int32, _L)) * _F

    def load_and_fire(f):
        iv = idxs[f % 2]
        ov = oidxs[f % 2]
        pltpu.sync_copy(xcft_hbm.at[pl.ds(f * _B + base, _BW)], iv)

        def addoff(i, carry2):
            sl = pl.ds(i * _L, _L)
            iv[sl] = iv[sl] + f * _VP
            ov[sl] = row0 + (i * _L * _F + f)
            return carry2

        lax.fori_loop(0, _BW // _L, addoff, 0, unroll=4)
        pltpu.async_copy(pk_hbm.at[iv], rows[f % 2], gsem)

    def drain_write():
        pltpu.make_async_copy(rows0, out_hbm.at[oidx0], wsem).wait()

    load_and_fire(0)
    for f in range(_F):
        if f + 1 < _F:
            if f >= 1:
                drain_write()
            load_and_fire(f + 1)
        pltpu.make_async_copy(pk_hbm.at[idxs[f % 2]], rows[f % 2], gsem).wait()
        pltpu.async_copy(rows[f % 2], out_hbm.at[oidxs[f % 2]], wsem)
    drain_write()
    drain_write()


def kernel(x_cont, x_cat, tables):
    tt = jnp.transpose(tables, (0, 2, 1))
    l32 = tables[:, _V - 32:, :].reshape(_F * 32 * _E)
    pk = _relayout_kernel(tt, l32)
    xcft = jnp.transpose(x_cat, (1, 0)).reshape(_F * _B)
    cat_bm = _gather_kernel(xcft, pk.reshape(_F * _VP, _E))
    return jnp.concatenate([x_cont, cat_bm.reshape(_B, _F * _E)], axis=1)
